# baseline (speedup 1.0000x reference)
import functools

import jax
import jax.numpy as jnp
from jax import lax
from jax.experimental import pallas as pl
from jax.experimental.pallas import tpu as pltpu
from jax.experimental.pallas import tpu_sc as plsc

NC, NS, LANES = 2, 16, 16
NW = NC * NS
K = 128
SCCH = 12
NBUF = 3
BN_EPS = 1e-5
BLK = 2000


ZR = 128


def _acc_rows(N):
    return -(-N // (NS * ZR)) * ZR


@functools.cache
def _seg_sum_kernel(N, D, CH):
    RPT = _acc_rows(N)
    NP = NS * RPT
    NSUP = CH // SCCH
    assert D % (2 * LANES) == 0 and K % LANES == 0 and CH % SCCH == 0
    assert SCCH % NBUF == 0

    mesh = plsc.VectorSubcoreMesh(core_axis_name="c", subcore_axis_name="s")

    @functools.partial(
        pl.kernel,
        out_type=jax.ShapeDtypeStruct((NC, NP, D), jnp.float32),
        mesh=mesh,
        compiler_params=pltpu.CompilerParams(use_tc_tiling_on_sc=False),
        scratch_types=[
            pltpu.VMEM((SCCH, K), jnp.int32),
            pltpu.VMEM((SCCH, K), jnp.int32),
            pltpu.VMEM((SCCH, K), jnp.float32),
            pltpu.VMEM((K, D // 2), jnp.int32),
            pltpu.VMEM((K, D // 2), jnp.int32),
            pltpu.VMEM((K, D // 2), jnp.int32),
            pltpu.VMEM((K, D), jnp.float32),
            pltpu.VMEM_SHARED((NP, D), jnp.float32),
            pltpu.SemaphoreType.DMA,
            pltpu.SemaphoreType.DMA,
            pltpu.SemaphoreType.DMA,
        ],
    )
    def seg(xb_hbm, src_hbm, dst_hbm, w_hbm, out_hbm,
            src_v, dst_v, w_v, rows0, rows1, rows2, frows, acc_s,
            sem0, sem1, sem2):
        c = lax.axis_index("c")
        s = lax.axis_index("s")
        wid = s * NC + c
        rows = (rows0, rows1, rows2)
        sems = (sem0, sem1, sem2)

        def zrow(i, carry):
            for j in range(D // LANES):
                frows[i, pl.ds(j * LANES, LANES)] = jnp.zeros((LANES,), jnp.float32)
            return carry
        lax.fori_loop(0, K, zrow, 0)
        for r in range(RPT // K):
            pltpu.sync_copy(frows, acc_s.at[pl.ds(s * RPT + r * K, K)])
        plsc.subcore_barrier()

        def gather_start(b, ch):
            pltpu.async_copy(xb_hbm.at[src_v.at[ch]], rows[b], sems[b])

        def gather_wait(b, ch):
            pltpu.make_async_copy(xb_hbm.at[src_v.at[ch]], rows[b], sems[b]).wait()

        def scale(buf, ch):
            def grp(g, carry):
                wv = w_v[ch, pl.ds(g * LANES, LANES)]
                for l in range(LANES):
                    w = wv[l]
                    k = g * LANES + l
                    for j in range(D // (2 * LANES)):
                        pk = buf[k, pl.ds(j * LANES, LANES)]
                        lo = lax.bitcast_convert_type(pk << 16, jnp.float32)
                        hi = lax.bitcast_convert_type(pk & jnp.int32(-65536),
                                                      jnp.float32)
                        frows[k, pl.ds(j * 2 * LANES, LANES)] = lo * w
                        frows[k, pl.ds(j * 2 * LANES + LANES, LANES)] = hi * w
                return carry
            lax.fori_loop(0, K // LANES, grp, 0)

        def super_chunk(sup, carry):
            ssl = pl.ds(sup * SCCH, SCCH)
            pltpu.sync_copy(src_hbm.at[wid, ssl], src_v)
            pltpu.sync_copy(dst_hbm.at[wid, ssl], dst_v)
            pltpu.sync_copy(w_hbm.at[wid, ssl], w_v)
            for b in range(NBUF):
                gather_start(b, b)

            def rotation(i, carry2):
                ch0 = NBUF * i
                for b in range(NBUF):
                    ch = ch0 + b
                    gather_wait(b, ch)
                    scale(rows[b], ch)
                    pltpu.sync_copy(frows, acc_s.at[dst_v.at[ch]], add=True)

                    @pl.when(ch + NBUF < SCCH)
                    def _():
                        gather_start(b, ch + NBUF)
                return carry2
            lax.fori_loop(0, SCCH // NBUF, rotation, 0)
            return carry
        lax.fori_loop(0, NSUP, super_chunk, 0)

        plsc.subcore_barrier()
        pltpu.sync_copy(acc_s.at[pl.ds(s * RPT, RPT)],
                        out_hbm.at[c, pl.ds(s * RPT, RPT)])

    return seg


def _dot_t(a, w):
    return lax.dot_general(a, w, (((1,), (1,)), ((), ())),
                           preferred_element_type=jnp.float32)


def _gate_body(p0_ref, p1_ref, x_ref,
               w1, u1, w2, u2, w3, u3, b1, b2, b3,
               out_ref, st_ref):
    i = pl.program_id(0)
    x = x_ref[...]
    agg = p0_ref[0] + p1_ref[0]
    ul = jax.nn.relu(_dot_t(agg, w1[...]) + _dot_t(x, u1[...]) + b1[...])
    rl = jax.nn.relu(_dot_t(agg, w2[...]) + _dot_t(x, u2[...]) + b2[...])
    fl = jnp.tanh(_dot_t(agg, w3[...]) + _dot_t(rl * x, u3[...]) + b3[...])
    out = ul * fl + (1.0 - ul) * x
    out_ref[...] = out

    @pl.when(i == 0)
    def _():
        st_ref[...] = jnp.zeros_like(st_ref)
    st_ref[0:1, :] += jnp.sum(out, axis=0, keepdims=True)
    st_ref[1:2, :] += jnp.sum(out * out, axis=0, keepdims=True)


def _bn_mlp_body(n_total, a_ref, st_in, g_ref, b_ref, w_ref, bias_ref,
                 out_ref, st_out):
    i = pl.program_id(0)
    mu = st_in[0:1, :] / n_total
    var = st_in[1:2, :] / n_total - mu * mu
    inv = lax.rsqrt(var + BN_EPS)
    a = (a_ref[...] - mu) * inv * g_ref[...] + b_ref[...]
    a = jax.nn.relu(a)
    h = _dot_t(a, w_ref[...]) + bias_ref[...]
    out_ref[...] = h

    @pl.when(i == 0)
    def _():
        st_out[...] = jnp.zeros_like(st_out)
    st_out[0:1, :] += jnp.sum(h, axis=0, keepdims=True)
    st_out[1:2, :] += jnp.sum(h * h, axis=0, keepdims=True)


def _bn_final_body(n_total, a_ref, st_in, g_ref, b_ref, w_ref, bias_ref,
                   out_ref):
    mu = st_in[0:1, :] / n_total
    var = st_in[1:2, :] / n_total - mu * mu
    inv = lax.rsqrt(var + BN_EPS)
    a = (a_ref[...] - mu) * inv * g_ref[...] + b_ref[...]
    a = jax.nn.relu(a)
    out_ref[...] = _dot_t(a, w_ref[...]) + bias_ref[...]


def _full(shape):
    return pl.BlockSpec(shape, lambda i: (0, 0))


def kernel(x, edge_index, edge_weight, W1_w, W1_b, U1_w, U1_b, W2_w, W2_b,
           U2_w, U2_b, W3_w, W3_b, U3_w, U3_b, bn_g, bn_b,
           m0_w, m0_b, mbn_g, mbn_b, m1_w, m1_b):
    N, D = x.shape
    E = edge_weight.shape[0]
    dst = edge_index[0]
    src = edge_index[1]

    e_w = -(-E // NW)
    CH = -(--(-e_w // K) // SCCH) * SCCH
    pad = NW * CH * K - E
    src_p = jnp.pad(src, (0, pad)).reshape(NW, CH, K)
    dst_p = jnp.pad(dst, (0, pad)).reshape(NW, CH, K)
    wgt_p = jnp.pad(edge_weight, (0, pad)).reshape(NW, CH, K)

    xb = (x.reshape(N, D // 32, 2, 16).transpose(0, 1, 3, 2)
          .reshape(N, D // 2, 2).astype(jnp.bfloat16))
    xb32 = lax.bitcast_convert_type(xb, jnp.int32)

    parts = _seg_sum_kernel(N, D, CH)(xb32, src_p, dst_p, wgt_p)

    grid = (N // BLK,)
    row_blk = pl.BlockSpec((BLK, D), lambda i: (i, 0))
    part0 = pl.BlockSpec((1, BLK, D), lambda i: (0, i, 0))
    part1 = pl.BlockSpec((1, BLK, D), lambda i: (1, i, 0))
    wspec = _full((D, D))
    vspec = _full((1, D))
    st_spec = _full((8, D))

    b1 = (W1_b + U1_b).reshape(1, D)
    b2 = (W2_b + U2_b).reshape(1, D)
    b3 = (W3_b + U3_b).reshape(1, D)

    out_pre, st1 = pl.pallas_call(
        _gate_body,
        grid=grid,
        in_specs=[part0, part1, row_blk] + [wspec] * 6 + [vspec] * 3,
        out_specs=[row_blk, st_spec],
        out_shape=[jax.ShapeDtypeStruct((N, D), jnp.float32),
                   jax.ShapeDtypeStruct((8, D), jnp.float32)],
    )(parts, parts, x, W1_w, U1_w, W2_w, U2_w, W3_w, U3_w, b1, b2, b3)

    h, st2 = pl.pallas_call(
        functools.partial(_bn_mlp_body, float(N)),
        grid=grid,
        in_specs=[row_blk, st_spec, vspec, vspec, wspec, vspec],
        out_specs=[row_blk, st_spec],
        out_shape=[jax.ShapeDtypeStruct((N, D), jnp.float32),
                   jax.ShapeDtypeStruct((8, D), jnp.float32)],
    )(out_pre, st1, bn_g.reshape(1, D), bn_b.reshape(1, D),
      m0_w, m0_b.reshape(1, D))

    y = pl.pallas_call(
        functools.partial(_bn_final_body, float(N)),
        grid=grid,
        in_specs=[row_blk, st_spec, vspec, vspec, wspec, vspec],
        out_specs=row_blk,
        out_shape=jax.ShapeDtypeStruct((N, D), jnp.float32),
    )(h, st2, mbn_g.reshape(1, D), mbn_b.reshape(1, D),
      m1_w, m1_b.reshape(1, D))

    return y


# speedup vs baseline: 1.5017x; 1.5017x over previous
import functools

import jax
import jax.numpy as jnp
from jax import lax
from jax.experimental import pallas as pl
from jax.experimental.pallas import tpu as pltpu
from jax.experimental.pallas import tpu_sc as plsc

NC, NS, LANES = 2, 16, 16
NW = NC * NS
K = 128
SCCH = 16
NBUF = 2
BN_EPS = 1e-5
BLK = 2000


ZR = 128


def _acc_rows(N):
    return -(-N // (NS * ZR)) * ZR


@functools.cache
def _seg_sum_kernel(N, D, CH):
    RPT = _acc_rows(N)
    NP = NS * RPT
    NSUP = CH // SCCH
    assert D % (2 * LANES) == 0 and K % LANES == 0 and CH % SCCH == 0
    assert SCCH % NBUF == 0

    mesh = plsc.VectorSubcoreMesh(core_axis_name="c", subcore_axis_name="s")

    @functools.partial(
        pl.kernel,
        out_type=jax.ShapeDtypeStruct((NC, NP, D), jnp.float32),
        mesh=mesh,
        compiler_params=pltpu.CompilerParams(use_tc_tiling_on_sc=False),
        scratch_types=[
            pltpu.VMEM((SCCH, K), jnp.int32),
            pltpu.VMEM((SCCH, K), jnp.int32),
            pltpu.VMEM((SCCH, K), jnp.float32),
            pltpu.VMEM((K, D // 2), jnp.int32),
            pltpu.VMEM((K, D // 2), jnp.int32),
            pltpu.VMEM((K, D), jnp.float32),
            pltpu.VMEM_SHARED((NP, D), jnp.float32),
            pltpu.SemaphoreType.DMA,
            pltpu.SemaphoreType.DMA,
        ],
    )
    def seg(xb_hbm, src_hbm, dst_hbm, w_hbm, out_hbm,
            src_v, dst_v, w_v, rows0, rows1, frows, acc_s, sem0, sem1):
        c = lax.axis_index("c")
        s = lax.axis_index("s")
        wid = s * NC + c
        rows = (rows0, rows1)
        sems = (sem0, sem1)

        def zrow(i, carry):
            for j in range(D // LANES):
                frows[i, pl.ds(j * LANES, LANES)] = jnp.zeros((LANES,), jnp.float32)
            return carry
        lax.fori_loop(0, K, zrow, 0)
        for r in range(RPT // K):
            pltpu.sync_copy(frows, acc_s.at[pl.ds(s * RPT + r * K, K)])
        plsc.subcore_barrier()

        def gather_start(b, ch):
            pltpu.async_copy(xb_hbm.at[src_v.at[ch]], rows[b], sems[b])

        def gather_wait(b, ch):
            pltpu.make_async_copy(xb_hbm.at[src_v.at[ch]], rows[b], sems[b]).wait()

        def scale(buf, ch):
            def grp(g, carry):
                wv = w_v[ch, pl.ds(g * LANES, LANES)]
                for l in range(LANES):
                    w = wv[l]
                    k = g * LANES + l
                    for j in range(D // (2 * LANES)):
                        pk = buf[k, pl.ds(j * LANES, LANES)]
                        lo = lax.bitcast_convert_type(pk << 16, jnp.float32)
                        hi = lax.bitcast_convert_type(pk & jnp.int32(-65536),
                                                      jnp.float32)
                        frows[k, pl.ds(j * 2 * LANES, LANES)] = lo * w
                        frows[k, pl.ds(j * 2 * LANES + LANES, LANES)] = hi * w
                return carry
            lax.fori_loop(0, K // LANES, grp, 0)

        def super_chunk(sup, carry):
            ssl = pl.ds(sup * SCCH, SCCH)
            pltpu.sync_copy(src_hbm.at[wid, ssl], src_v)
            pltpu.sync_copy(dst_hbm.at[wid, ssl], dst_v)
            pltpu.sync_copy(w_hbm.at[wid, ssl], w_v)
            for b in range(NBUF):
                gather_start(b, b)

            def rotation(i, carry2):
                ch0 = NBUF * i
                for b in range(NBUF):
                    ch = ch0 + b
                    gather_wait(b, ch)
                    scale(rows[b], ch)
                    pltpu.sync_copy(frows, acc_s.at[dst_v.at[ch]], add=True)

                    @pl.when(ch + NBUF < SCCH)
                    def _():
                        gather_start(b, ch + NBUF)
                return carry2
            lax.fori_loop(0, SCCH // NBUF, rotation, 0)
            return carry
        lax.fori_loop(0, NSUP, super_chunk, 0)

        plsc.subcore_barrier()
        pltpu.sync_copy(acc_s.at[pl.ds(s * RPT, RPT)],
                        out_hbm.at[c, pl.ds(s * RPT, RPT)])

    return seg


def _dot_t(a, w):
    return lax.dot_general(a, w, (((1,), (1,)), ((), ())),
                           preferred_element_type=jnp.float32)


def _gate_body(p0_ref, p1_ref, x_ref,
               w1, u1, w2, u2, w3, u3, b1, b2, b3,
               out_ref, st_ref):
    i = pl.program_id(0)
    x = x_ref[...]
    agg = p0_ref[0] + p1_ref[0]
    ul = jax.nn.relu(_dot_t(agg, w1[...]) + _dot_t(x, u1[...]) + b1[...])
    rl = jax.nn.relu(_dot_t(agg, w2[...]) + _dot_t(x, u2[...]) + b2[...])
    fl = jnp.tanh(_dot_t(agg, w3[...]) + _dot_t(rl * x, u3[...]) + b3[...])
    out = ul * fl + (1.0 - ul) * x
    out_ref[...] = out

    @pl.when(i == 0)
    def _():
        st_ref[...] = jnp.zeros_like(st_ref)
    st_ref[0:1, :] += jnp.sum(out, axis=0, keepdims=True)
    st_ref[1:2, :] += jnp.sum(out * out, axis=0, keepdims=True)


def _bn_mlp_body(n_total, a_ref, st_in, g_ref, b_ref, w_ref, bias_ref,
                 out_ref, st_out):
    i = pl.program_id(0)
    mu = st_in[0:1, :] / n_total
    var = st_in[1:2, :] / n_total - mu * mu
    inv = lax.rsqrt(var + BN_EPS)
    a = (a_ref[...] - mu) * inv * g_ref[...] + b_ref[...]
    a = jax.nn.relu(a)
    h = _dot_t(a, w_ref[...]) + bias_ref[...]
    out_ref[...] = h

    @pl.when(i == 0)
    def _():
        st_out[...] = jnp.zeros_like(st_out)
    st_out[0:1, :] += jnp.sum(h, axis=0, keepdims=True)
    st_out[1:2, :] += jnp.sum(h * h, axis=0, keepdims=True)


def _bn_final_body(n_total, a_ref, st_in, g_ref, b_ref, w_ref, bias_ref,
                   out_ref):
    mu = st_in[0:1, :] / n_total
    var = st_in[1:2, :] / n_total - mu * mu
    inv = lax.rsqrt(var + BN_EPS)
    a = (a_ref[...] - mu) * inv * g_ref[...] + b_ref[...]
    a = jax.nn.relu(a)
    out_ref[...] = _dot_t(a, w_ref[...]) + bias_ref[...]


def _full(shape):
    return pl.BlockSpec(shape, lambda i: (0, 0))


def kernel(x, edge_index, edge_weight, W1_w, W1_b, U1_w, U1_b, W2_w, W2_b,
           U2_w, U2_b, W3_w, W3_b, U3_w, U3_b, bn_g, bn_b,
           m0_w, m0_b, mbn_g, mbn_b, m1_w, m1_b):
    N, D = x.shape
    E = edge_weight.shape[0]
    dst = edge_index[0]
    src = edge_index[1]

    e_w = -(-E // NW)
    CH = -(--(-e_w // K) // SCCH) * SCCH
    pad = NW * CH * K - E
    src_p = jnp.pad(src, (0, pad)).reshape(NW, CH, K)
    dst_p = jnp.pad(dst, (0, pad)).reshape(NW, CH, K)
    wgt_p = jnp.pad(edge_weight, (0, pad)).reshape(NW, CH, K)

    xb = (x.reshape(N, D // 32, 2, 16).transpose(0, 1, 3, 2)
          .reshape(N, D // 2, 2).astype(jnp.bfloat16))
    xb32 = lax.bitcast_convert_type(xb, jnp.int32)

    parts = _seg_sum_kernel(N, D, CH)(xb32, src_p, dst_p, wgt_p)

    grid = (N // BLK,)
    row_blk = pl.BlockSpec((BLK, D), lambda i: (i, 0))
    part0 = pl.BlockSpec((1, BLK, D), lambda i: (0, i, 0))
    part1 = pl.BlockSpec((1, BLK, D), lambda i: (1, i, 0))
    wspec = _full((D, D))
    vspec = _full((1, D))
    st_spec = _full((8, D))

    b1 = (W1_b + U1_b).reshape(1, D)
    b2 = (W2_b + U2_b).reshape(1, D)
    b3 = (W3_b + U3_b).reshape(1, D)

    out_pre, st1 = pl.pallas_call(
        _gate_body,
        grid=grid,
        in_specs=[part0, part1, row_blk] + [wspec] * 6 + [vspec] * 3,
        out_specs=[row_blk, st_spec],
        out_shape=[jax.ShapeDtypeStruct((N, D), jnp.float32),
                   jax.ShapeDtypeStruct((8, D), jnp.float32)],
    )(parts, parts, x, W1_w, U1_w, W2_w, U2_w, W3_w, U3_w, b1, b2, b3)

    h, st2 = pl.pallas_call(
        functools.partial(_bn_mlp_body, float(N)),
        grid=grid,
        in_specs=[row_blk, st_spec, vspec, vspec, wspec, vspec],
        out_specs=[row_blk, st_spec],
        out_shape=[jax.ShapeDtypeStruct((N, D), jnp.float32),
                   jax.ShapeDtypeStruct((8, D), jnp.float32)],
    )(out_pre, st1, bn_g.reshape(1, D), bn_b.reshape(1, D),
      m0_w, m0_b.reshape(1, D))

    y = pl.pallas_call(
        functools.partial(_bn_final_body, float(N)),
        grid=grid,
        in_specs=[row_blk, st_spec, vspec, vspec, wspec, vspec],
        out_specs=row_blk,
        out_shape=jax.ShapeDtypeStruct((N, D), jnp.float32),
    )(h, st2, mbn_g.reshape(1, D), mbn_b.reshape(1, D),
      m1_w, m1_b.reshape(1, D))

    return y


# speedup vs baseline: 1.5128x; 1.0074x over previous
import functools

import jax
import jax.numpy as jnp
from jax import lax
from jax.experimental import pallas as pl
from jax.experimental.pallas import tpu as pltpu
from jax.experimental.pallas import tpu_sc as plsc

NC, NS, LANES = 2, 16, 16
NW = NC * NS
K = 128
SCCH = 16
NBUF = 2
BN_EPS = 1e-5
BLK = 2000


ZR = 128


def _acc_rows(N):
    return -(-N // (NS * ZR)) * ZR


@functools.cache
def _seg_sum_kernel(N, D, CH):
    RPT = _acc_rows(N)
    NP = NS * RPT
    NSUP = CH // SCCH
    assert D % (2 * LANES) == 0 and K % LANES == 0 and CH % SCCH == 0
    assert SCCH % NBUF == 0

    mesh = plsc.VectorSubcoreMesh(core_axis_name="c", subcore_axis_name="s")

    @functools.partial(
        pl.kernel,
        out_type=jax.ShapeDtypeStruct((NC, NP, D), jnp.float32),
        mesh=mesh,
        compiler_params=pltpu.CompilerParams(use_tc_tiling_on_sc=False),
        scratch_types=[
            pltpu.VMEM((SCCH, K), jnp.int32),
            pltpu.VMEM((SCCH, K), jnp.int32),
            pltpu.VMEM((SCCH, K), jnp.float32),
            pltpu.VMEM((K, D // 2), jnp.int32),
            pltpu.VMEM((K, D // 2), jnp.int32),
            pltpu.VMEM((K, D), jnp.float32),
            pltpu.VMEM_SHARED((NP, D), jnp.float32),
            pltpu.SemaphoreType.DMA,
            pltpu.SemaphoreType.DMA,
        ],
    )
    def seg(xb_hbm, src_hbm, dst_hbm, w_hbm, out_hbm,
            src_v, dst_v, w_v, rows0, rows1, frows, acc_s, sem0, sem1):
        c = lax.axis_index("c")
        s = lax.axis_index("s")
        wid = s * NC + c
        rows = (rows0, rows1)
        sems = (sem0, sem1)

        def zrow(i, carry):
            for j in range(D // LANES):
                frows[i, pl.ds(j * LANES, LANES)] = jnp.zeros((LANES,), jnp.float32)
            return carry
        lax.fori_loop(0, K, zrow, 0)
        for r in range(RPT // K):
            pltpu.sync_copy(frows, acc_s.at[pl.ds(s * RPT + r * K, K)])
        plsc.subcore_barrier()

        def gather_start(b, ch):
            h = K // 2
            pltpu.async_copy(xb_hbm.at[src_v.at[ch, pl.ds(0, h)]],
                             rows[b].at[pl.ds(0, h)], sems[b])
            pltpu.async_copy(xb_hbm.at[src_v.at[ch, pl.ds(h, h)]],
                             rows[b].at[pl.ds(h, h)], sems[b])

        def gather_wait(b, ch):
            pltpu.make_async_copy(xb_hbm.at[src_v.at[ch]], rows[b], sems[b]).wait()

        def scale(buf, ch):
            def grp(g, carry):
                wv = w_v[ch, pl.ds(g * LANES, LANES)]
                for l in range(LANES):
                    w = wv[l]
                    k = g * LANES + l
                    for j in range(D // (2 * LANES)):
                        pk = buf[k, pl.ds(j * LANES, LANES)]
                        lo = lax.bitcast_convert_type(pk << 16, jnp.float32)
                        hi = lax.bitcast_convert_type(pk & jnp.int32(-65536),
                                                      jnp.float32)
                        frows[k, pl.ds(j * 2 * LANES, LANES)] = lo * w
                        frows[k, pl.ds(j * 2 * LANES + LANES, LANES)] = hi * w
                return carry
            lax.fori_loop(0, K // LANES, grp, 0)

        def super_chunk(sup, carry):
            ssl = pl.ds(sup * SCCH, SCCH)
            pltpu.sync_copy(src_hbm.at[wid, ssl], src_v)
            pltpu.sync_copy(dst_hbm.at[wid, ssl], dst_v)
            pltpu.sync_copy(w_hbm.at[wid, ssl], w_v)
            for b in range(NBUF):
                gather_start(b, b)

            def rotation(i, carry2):
                ch0 = NBUF * i
                for b in range(NBUF):
                    ch = ch0 + b
                    gather_wait(b, ch)
                    scale(rows[b], ch)
                    pltpu.sync_copy(frows, acc_s.at[dst_v.at[ch]], add=True)

                    @pl.when(ch + NBUF < SCCH)
                    def _():
                        gather_start(b, ch + NBUF)
                return carry2
            lax.fori_loop(0, SCCH // NBUF, rotation, 0)
            return carry
        lax.fori_loop(0, NSUP, super_chunk, 0)

        plsc.subcore_barrier()
        pltpu.sync_copy(acc_s.at[pl.ds(s * RPT, RPT)],
                        out_hbm.at[c, pl.ds(s * RPT, RPT)])

    return seg


def _dot_t(a, w):
    return lax.dot_general(a, w, (((1,), (1,)), ((), ())),
                           preferred_element_type=jnp.float32)


def _gate_body(p0_ref, p1_ref, x_ref,
               w1, u1, w2, u2, w3, u3, b1, b2, b3,
               out_ref, st_ref):
    i = pl.program_id(0)
    x = x_ref[...]
    agg = p0_ref[0] + p1_ref[0]
    ul = jax.nn.relu(_dot_t(agg, w1[...]) + _dot_t(x, u1[...]) + b1[...])
    rl = jax.nn.relu(_dot_t(agg, w2[...]) + _dot_t(x, u2[...]) + b2[...])
    fl = jnp.tanh(_dot_t(agg, w3[...]) + _dot_t(rl * x, u3[...]) + b3[...])
    out = ul * fl + (1.0 - ul) * x
    out_ref[...] = out

    @pl.when(i == 0)
    def _():
        st_ref[...] = jnp.zeros_like(st_ref)
    st_ref[0:1, :] += jnp.sum(out, axis=0, keepdims=True)
    st_ref[1:2, :] += jnp.sum(out * out, axis=0, keepdims=True)


def _bn_mlp_body(n_total, a_ref, st_in, g_ref, b_ref, w_ref, bias_ref,
                 out_ref, st_out):
    i = pl.program_id(0)
    mu = st_in[0:1, :] / n_total
    var = st_in[1:2, :] / n_total - mu * mu
    inv = lax.rsqrt(var + BN_EPS)
    a = (a_ref[...] - mu) * inv * g_ref[...] + b_ref[...]
    a = jax.nn.relu(a)
    h = _dot_t(a, w_ref[...]) + bias_ref[...]
    out_ref[...] = h

    @pl.when(i == 0)
    def _():
        st_out[...] = jnp.zeros_like(st_out)
    st_out[0:1, :] += jnp.sum(h, axis=0, keepdims=True)
    st_out[1:2, :] += jnp.sum(h * h, axis=0, keepdims=True)


def _bn_final_body(n_total, a_ref, st_in, g_ref, b_ref, w_ref, bias_ref,
                   out_ref):
    mu = st_in[0:1, :] / n_total
    var = st_in[1:2, :] / n_total - mu * mu
    inv = lax.rsqrt(var + BN_EPS)
    a = (a_ref[...] - mu) * inv * g_ref[...] + b_ref[...]
    a = jax.nn.relu(a)
    out_ref[...] = _dot_t(a, w_ref[...]) + bias_ref[...]


def _full(shape):
    return pl.BlockSpec(shape, lambda i: (0, 0))


def kernel(x, edge_index, edge_weight, W1_w, W1_b, U1_w, U1_b, W2_w, W2_b,
           U2_w, U2_b, W3_w, W3_b, U3_w, U3_b, bn_g, bn_b,
           m0_w, m0_b, mbn_g, mbn_b, m1_w, m1_b):
    N, D = x.shape
    E = edge_weight.shape[0]
    dst = edge_index[0]
    src = edge_index[1]

    e_w = -(-E // NW)
    CH = -(--(-e_w // K) // SCCH) * SCCH
    pad = NW * CH * K - E
    src_p = jnp.pad(src, (0, pad)).reshape(NW, CH, K)
    dst_p = jnp.pad(dst, (0, pad)).reshape(NW, CH, K)
    wgt_p = jnp.pad(edge_weight, (0, pad)).reshape(NW, CH, K)

    xb = (x.reshape(N, D // 32, 2, 16).transpose(0, 1, 3, 2)
          .reshape(N, D // 2, 2).astype(jnp.bfloat16))
    xb32 = lax.bitcast_convert_type(xb, jnp.int32)

    parts = _seg_sum_kernel(N, D, CH)(xb32, src_p, dst_p, wgt_p)

    grid = (N // BLK,)
    row_blk = pl.BlockSpec((BLK, D), lambda i: (i, 0))
    part0 = pl.BlockSpec((1, BLK, D), lambda i: (0, i, 0))
    part1 = pl.BlockSpec((1, BLK, D), lambda i: (1, i, 0))
    wspec = _full((D, D))
    vspec = _full((1, D))
    st_spec = _full((8, D))

    b1 = (W1_b + U1_b).reshape(1, D)
    b2 = (W2_b + U2_b).reshape(1, D)
    b3 = (W3_b + U3_b).reshape(1, D)

    out_pre, st1 = pl.pallas_call(
        _gate_body,
        grid=grid,
        in_specs=[part0, part1, row_blk] + [wspec] * 6 + [vspec] * 3,
        out_specs=[row_blk, st_spec],
        out_shape=[jax.ShapeDtypeStruct((N, D), jnp.float32),
                   jax.ShapeDtypeStruct((8, D), jnp.float32)],
    )(parts, parts, x, W1_w, U1_w, W2_w, U2_w, W3_w, U3_w, b1, b2, b3)

    h, st2 = pl.pallas_call(
        functools.partial(_bn_mlp_body, float(N)),
        grid=grid,
        in_specs=[row_blk, st_spec, vspec, vspec, wspec, vspec],
        out_specs=[row_blk, st_spec],
        out_shape=[jax.ShapeDtypeStruct((N, D), jnp.float32),
                   jax.ShapeDtypeStruct((8, D), jnp.float32)],
    )(out_pre, st1, bn_g.reshape(1, D), bn_b.reshape(1, D),
      m0_w, m0_b.reshape(1, D))

    y = pl.pallas_call(
        functools.partial(_bn_final_body, float(N)),
        grid=grid,
        in_specs=[row_blk, st_spec, vspec, vspec, wspec, vspec],
        out_specs=row_blk,
        out_shape=jax.ShapeDtypeStruct((N, D), jnp.float32),
    )(h, st2, mbn_g.reshape(1, D), mbn_b.reshape(1, D),
      m1_w, m1_b.reshape(1, D))

    return y
